# baseline (speedup 1.0000x reference)
import functools

import jax
import jax.numpy as jnp
from jax import lax
from jax.experimental import pallas as pl
from jax.experimental.pallas import tpu as pltpu
from jax.experimental.pallas import tpu_sc as plsc

_IDX_MINOR = 128


@functools.lru_cache(maxsize=None)
def _make_sc_gather(n_rel, emb2, batch):
    info = plsc.get_sparse_core_info()
    nc, ns = info.num_cores, info.num_subcores
    nw = nc * ns
    assert batch % (nw * _IDX_MINOR) == 0
    chunks = batch // (nw * _IDX_MINOR)
    b_per_w = chunks * _IDX_MINOR

    mesh = plsc.VectorSubcoreMesh(core_axis_name="c", subcore_axis_name="s")

    @functools.partial(
        pl.kernel,
        out_type=jax.ShapeDtypeStruct((batch, emb2), jnp.float32),
        mesh=mesh,
        scratch_types=[
            pltpu.VMEM((chunks, _IDX_MINOR), jnp.int32),
            pltpu.VMEM((b_per_w, emb2), jnp.float32),
            pltpu.SemaphoreType.DMA,
        ],
        compiler_params=pltpu.CompilerParams(use_tc_tiling_on_sc=False),
    )
    def gather(table_hbm, idx_hbm, out_hbm, idx_v, rows_v, sem):
        wid = lax.axis_index("s") * nc + lax.axis_index("c")
        pltpu.sync_copy(idx_hbm.at[pl.ds(wid * chunks, chunks)], idx_v)
        copies = []
        for j in range(chunks):
            copies.append(
                pltpu.async_copy(
                    table_hbm.at[idx_v.at[j]],
                    rows_v.at[pl.ds(j * _IDX_MINOR, _IDX_MINOR)],
                    sem,
                )
            )
        for c in copies:
            c.wait()
        pltpu.sync_copy(rows_v, out_hbm.at[pl.ds(wid * b_per_w, b_per_w)])

    return gather


def _proj_body_t(x_ref, r_ref, o_ref):
    x = x_ref[...]
    r = r_ref[...][None, :, :]
    prod = x * r
    p = jnp.sum(prod, axis=1, keepdims=True)
    o_ref[...] = x - p * r


def kernel(node_emb, relation, rela_emb):
    batch, hist, emb = node_emb.shape
    idx = relation.astype(jnp.int32).reshape(batch // _IDX_MINOR, _IDX_MINOR)
    table2 = jnp.concatenate([rela_emb, rela_emb], axis=-1)

    r2 = _make_sc_gather(rela_emb.shape[0], 2 * emb, batch)(table2, idx)
    r_t = jnp.transpose(r2)

    x_t = jnp.transpose(node_emb, (1, 2, 0))

    bb = 256
    out_t = pl.pallas_call(
        _proj_body_t,
        grid=(batch // bb,),
        in_specs=[
            pl.BlockSpec((hist, emb, bb), lambda i: (0, 0, i)),
            pl.BlockSpec((emb, bb), lambda i: (0, i)),
        ],
        out_specs=pl.BlockSpec((hist, emb, bb), lambda i: (0, 0, i)),
        out_shape=jax.ShapeDtypeStruct((hist, emb, batch), jnp.float32),
    )(x_t, r_t)
    return jnp.transpose(out_t, (2, 0, 1))


# speedup vs baseline: 1.0341x; 1.0341x over previous
import functools

import jax
import jax.numpy as jnp
from jax import lax
from jax.experimental import pallas as pl
from jax.experimental.pallas import tpu as pltpu
from jax.experimental.pallas import tpu_sc as plsc

_IDX_MINOR = 128


@functools.lru_cache(maxsize=None)
def _make_sc_gather(n_rel, emb2, batch):
    info = plsc.get_sparse_core_info()
    nc, ns = info.num_cores, info.num_subcores
    nw = nc * ns
    assert batch % (nw * _IDX_MINOR) == 0
    chunks = batch // (nw * _IDX_MINOR)
    b_per_w = chunks * _IDX_MINOR

    mesh = plsc.VectorSubcoreMesh(core_axis_name="c", subcore_axis_name="s")

    @functools.partial(
        pl.kernel,
        out_type=jax.ShapeDtypeStruct((batch, emb2), jnp.float32),
        mesh=mesh,
        scratch_types=[
            pltpu.VMEM((chunks, _IDX_MINOR), jnp.int32),
            pltpu.VMEM((b_per_w, emb2), jnp.float32),
            pltpu.SemaphoreType.DMA,
        ],
        compiler_params=pltpu.CompilerParams(use_tc_tiling_on_sc=False),
    )
    def gather(table_hbm, idx_hbm, out_hbm, idx_v, rows_v, sem):
        wid = lax.axis_index("s") * nc + lax.axis_index("c")
        pltpu.sync_copy(idx_hbm.at[pl.ds(wid * chunks, chunks)], idx_v)
        copies = []
        for j in range(chunks):
            copies.append(
                pltpu.async_copy(
                    table_hbm.at[idx_v.at[j]],
                    rows_v.at[pl.ds(j * _IDX_MINOR, _IDX_MINOR)],
                    sem,
                )
            )
        for c in copies:
            c.wait()
        pltpu.sync_copy(rows_v, out_hbm.at[pl.ds(wid * b_per_w, b_per_w)])

    return gather


def _proj_body_t(x_ref, r_ref, o_ref):
    x = x_ref[...]
    r = r_ref[...][None, :, :]
    prod = x * r
    p = jnp.sum(prod, axis=1, keepdims=True)
    o_ref[...] = x - p * r


def kernel(node_emb, relation, rela_emb):
    batch, hist, emb = node_emb.shape
    idx = relation.astype(jnp.int32).reshape(batch // _IDX_MINOR, _IDX_MINOR)
    table2 = jnp.concatenate([rela_emb, rela_emb], axis=-1)

    r2 = _make_sc_gather(rela_emb.shape[0], 2 * emb, batch)(table2, idx)
    r_t = jnp.transpose(r2)

    x_t = jnp.transpose(node_emb, (1, 2, 0))

    bb = 1024
    out_t = pl.pallas_call(
        _proj_body_t,
        grid=(batch // bb,),
        in_specs=[
            pl.BlockSpec((hist, emb, bb), lambda i: (0, 0, i)),
            pl.BlockSpec((emb, bb), lambda i: (0, i)),
        ],
        out_specs=pl.BlockSpec((hist, emb, bb), lambda i: (0, 0, i)),
        out_shape=jax.ShapeDtypeStruct((hist, emb, batch), jnp.float32),
    )(x_t, r_t)
    return jnp.transpose(out_t, (2, 0, 1))


# speedup vs baseline: 1.0730x; 1.0377x over previous
import functools

import jax
import jax.numpy as jnp
from jax import lax
from jax.experimental import pallas as pl
from jax.experimental.pallas import tpu as pltpu
from jax.experimental.pallas import tpu_sc as plsc

_IDX_MINOR = 128


@functools.lru_cache(maxsize=None)
def _make_sc_gather(n_rel, emb2, batch):
    info = plsc.get_sparse_core_info()
    nc, ns = info.num_cores, info.num_subcores
    nw = nc * ns
    assert batch % (nw * _IDX_MINOR) == 0
    chunks = batch // (nw * _IDX_MINOR)
    b_per_w = chunks * _IDX_MINOR

    mesh = plsc.VectorSubcoreMesh(core_axis_name="c", subcore_axis_name="s")

    @functools.partial(
        pl.kernel,
        out_type=jax.ShapeDtypeStruct((batch, emb2), jnp.float32),
        mesh=mesh,
        scratch_types=[
            pltpu.VMEM((chunks, _IDX_MINOR), jnp.int32),
            pltpu.VMEM((b_per_w, emb2), jnp.float32),
            pltpu.SemaphoreType.DMA,
        ],
        compiler_params=pltpu.CompilerParams(use_tc_tiling_on_sc=False),
    )
    def gather(table_hbm, idx_hbm, out_hbm, idx_v, rows_v, sem):
        wid = lax.axis_index("s") * nc + lax.axis_index("c")
        pltpu.sync_copy(idx_hbm.at[pl.ds(wid * chunks, chunks)], idx_v)
        copies = []
        for j in range(chunks):
            copies.append(
                pltpu.async_copy(
                    table_hbm.at[idx_v.at[j]],
                    rows_v.at[pl.ds(j * _IDX_MINOR, _IDX_MINOR)],
                    sem,
                )
            )
        for c in copies:
            c.wait()
        pltpu.sync_copy(rows_v, out_hbm.at[pl.ds(wid * b_per_w, b_per_w)])

    return gather


def _proj_body_t(x_ref, r_ref, o_ref):
    x = x_ref[...]
    r = jnp.transpose(r_ref[...][:, :64])[None, :, :]
    prod = x * r
    p = jnp.sum(prod, axis=1, keepdims=True)
    o_ref[...] = x - p * r


def kernel(node_emb, relation, rela_emb):
    batch, hist, emb = node_emb.shape
    idx = relation.astype(jnp.int32).reshape(batch // _IDX_MINOR, _IDX_MINOR)
    table2 = jnp.concatenate([rela_emb, rela_emb], axis=-1)

    r2 = _make_sc_gather(rela_emb.shape[0], 2 * emb, batch)(table2, idx)

    x_t = jnp.transpose(node_emb, (1, 2, 0))

    bb = 512
    out_t = pl.pallas_call(
        _proj_body_t,
        grid=(batch // bb,),
        in_specs=[
            pl.BlockSpec((hist, emb, bb), lambda i: (0, 0, i)),
            pl.BlockSpec((bb, 2 * emb), lambda i: (i, 0)),
        ],
        out_specs=pl.BlockSpec((hist, emb, bb), lambda i: (0, 0, i)),
        out_shape=jax.ShapeDtypeStruct((hist, emb, batch), jnp.float32),
    )(x_t, r2)
    return jnp.transpose(out_t, (2, 0, 1))


# speedup vs baseline: 1.0799x; 1.0065x over previous
import functools

import jax
import jax.numpy as jnp
from jax import lax
from jax.experimental import pallas as pl
from jax.experimental.pallas import tpu as pltpu
from jax.experimental.pallas import tpu_sc as plsc

_IDX_MINOR = 128


@functools.lru_cache(maxsize=None)
def _make_sc_gather(n_rel, emb2, batch):
    info = plsc.get_sparse_core_info()
    nc, ns = info.num_cores, info.num_subcores
    nw = nc * ns
    assert batch % (nw * _IDX_MINOR) == 0
    chunks = batch // (nw * _IDX_MINOR)
    b_per_w = chunks * _IDX_MINOR

    mesh = plsc.VectorSubcoreMesh(core_axis_name="c", subcore_axis_name="s")

    @functools.partial(
        pl.kernel,
        out_type=jax.ShapeDtypeStruct((batch, emb2), jnp.float32),
        mesh=mesh,
        scratch_types=[
            pltpu.VMEM((chunks, _IDX_MINOR), jnp.int32),
            pltpu.VMEM((b_per_w, emb2), jnp.float32),
            pltpu.SemaphoreType.DMA,
        ],
        compiler_params=pltpu.CompilerParams(use_tc_tiling_on_sc=False),
    )
    def gather(table_hbm, idx_hbm, out_hbm, idx_v, rows_v, sem):
        wid = lax.axis_index("s") * nc + lax.axis_index("c")
        pltpu.sync_copy(idx_hbm.at[pl.ds(wid * chunks, chunks)], idx_v)
        copies = []
        for j in range(chunks):
            copies.append(
                pltpu.async_copy(
                    table_hbm.at[idx_v.at[j]],
                    rows_v.at[pl.ds(j * _IDX_MINOR, _IDX_MINOR)],
                    sem,
                )
            )
        for c in copies:
            c.wait()
        pltpu.sync_copy(rows_v, out_hbm.at[pl.ds(wid * b_per_w, b_per_w)])

    return gather


def _proj_body_t(x_ref, r_ref, o_ref):
    x = x_ref[...]
    r = jnp.transpose(r_ref[...][:, :64])[None, :, :]
    prod = x * r
    p = jnp.sum(prod, axis=1, keepdims=True)
    o_ref[...] = x - p * r


def kernel(node_emb, relation, rela_emb):
    batch, hist, emb = node_emb.shape
    idx = relation.astype(jnp.int32).reshape(batch // _IDX_MINOR, _IDX_MINOR)
    table2 = jnp.concatenate([rela_emb, rela_emb], axis=-1)

    r2 = _make_sc_gather(rela_emb.shape[0], 2 * emb, batch)(table2, idx)

    x_t = jnp.transpose(node_emb, (1, 2, 0))

    bb = 1024
    out_t = pl.pallas_call(
        _proj_body_t,
        grid=(batch // bb,),
        in_specs=[
            pl.BlockSpec((hist, emb, bb), lambda i: (0, 0, i)),
            pl.BlockSpec((bb, 2 * emb), lambda i: (i, 0)),
        ],
        out_specs=pl.BlockSpec((hist, emb, bb), lambda i: (0, 0, i)),
        out_shape=jax.ShapeDtypeStruct((hist, emb, batch), jnp.float32),
        compiler_params=pltpu.CompilerParams(
            vmem_limit_bytes=100 * 1024 * 1024
        ),
    )(x_t, r2)
    return jnp.transpose(out_t, (2, 0, 1))


# speedup vs baseline: 1.0805x; 1.0005x over previous
import functools

import jax
import jax.numpy as jnp
from jax import lax
from jax.experimental import pallas as pl
from jax.experimental.pallas import tpu as pltpu
from jax.experimental.pallas import tpu_sc as plsc

_IDX_MINOR = 128


@functools.lru_cache(maxsize=None)
def _make_sc_gather(n_rel, emb2, batch):
    info = plsc.get_sparse_core_info()
    nc, ns = info.num_cores, info.num_subcores
    nw = nc * ns
    assert batch % (nw * _IDX_MINOR) == 0
    chunks = batch // (nw * _IDX_MINOR)
    b_per_w = chunks * _IDX_MINOR

    mesh = plsc.VectorSubcoreMesh(core_axis_name="c", subcore_axis_name="s")

    @functools.partial(
        pl.kernel,
        out_type=jax.ShapeDtypeStruct((batch, emb2), jnp.float32),
        mesh=mesh,
        scratch_types=[
            pltpu.VMEM((chunks, _IDX_MINOR), jnp.int32),
            pltpu.VMEM((b_per_w, emb2), jnp.float32),
            pltpu.SemaphoreType.DMA,
        ],
        compiler_params=pltpu.CompilerParams(use_tc_tiling_on_sc=False),
    )
    def gather(table_hbm, idx_hbm, out_hbm, idx_v, rows_v, sem):
        wid = lax.axis_index("s") * nc + lax.axis_index("c")
        pltpu.sync_copy(idx_hbm.at[pl.ds(wid * chunks, chunks)], idx_v)
        copies = []
        for j in range(chunks):
            copies.append(
                pltpu.async_copy(
                    table_hbm.at[idx_v.at[j]],
                    rows_v.at[pl.ds(j * _IDX_MINOR, _IDX_MINOR)],
                    sem,
                )
            )
        for c in copies:
            c.wait()
        pltpu.sync_copy(rows_v, out_hbm.at[pl.ds(wid * b_per_w, b_per_w)])

    return gather


def _proj_body_t(x_ref, r_ref, o_ref):
    x = x_ref[...]
    r = jnp.transpose(r_ref[...][:, : x.shape[1]])[None, :, :]
    prod = x * r
    p = jnp.sum(prod, axis=1, keepdims=True)
    o_ref[...] = x - p * r


def kernel(node_emb, relation, rela_emb):
    batch, hist, emb = node_emb.shape
    idx = relation.astype(jnp.int32).reshape(batch // _IDX_MINOR, _IDX_MINOR)
    table2 = jnp.concatenate([rela_emb, rela_emb], axis=-1)

    r2 = _make_sc_gather(rela_emb.shape[0], 2 * emb, batch)(table2, idx)

    x_t = jnp.transpose(node_emb, (1, 2, 0))

    bb = 1024
    out_t = pl.pallas_call(
        _proj_body_t,
        grid=(batch // bb,),
        in_specs=[
            pl.BlockSpec((hist, emb, bb), lambda i: (0, 0, i)),
            pl.BlockSpec((bb, 2 * emb), lambda i: (i, 0)),
        ],
        out_specs=pl.BlockSpec((hist, emb, bb), lambda i: (0, 0, i)),
        out_shape=jax.ShapeDtypeStruct((hist, emb, batch), jnp.float32),
        compiler_params=pltpu.CompilerParams(
            vmem_limit_bytes=100 * 1024 * 1024
        ),
    )(x_t, r2)
    return jnp.transpose(out_t, (2, 0, 1))
